# Initial kernel scaffold; baseline (speedup 1.0000x reference)
import jax
import jax.numpy as jnp
from jax.experimental import pallas as pl


def kernel(embed, edge_index, edge_weight):
    raise NotImplementedError("write your pallas kernel here")


# speedup vs baseline: 4.0680x; 4.0680x over previous
import functools

import jax
import jax.numpy as jnp
from jax import lax
from jax.experimental import pallas as pl
from jax.experimental.pallas import tpu as pltpu
from jax.experimental.pallas import tpu_sc as plsc

N_NODES = 100000
EMBED_DIM = 32
HALF_DIM = 16
N_EDGES = 1600000

EPB = 128
N_BLOCKS = N_EDGES // EPB
NS = 16
CPT = -(-N_BLOCKS // NS)
ACC_ROWS = 100096
N_ZB = ACC_ROWS // EPB
ZBPT = -(-N_ZB // NS)
DBLK = 400
N_DB = N_NODES // DBLK
DBPT = -(-N_DB // NS)


def _zero_rowsb(rowsb):
    def zb(e, c):
        rowsb[e, :] = jnp.zeros((HALF_DIM,), jnp.float32)
        return c
    lax.fori_loop(0, EPB, zb, 0, unroll=8)


def _layer(cur, out, src, dst, w, tid, srcb, dstb, wb, rowsb, acc, sem):
    _zero_rowsb(rowsb)
    nzb = jnp.minimum(ZBPT, jnp.maximum(0, N_ZB - tid * ZBPT))

    def zcopy(i, c):
        r0 = pl.multiple_of((tid * ZBPT + i) * EPB, 8)
        pltpu.sync_copy(rowsb, acc.at[pl.ds(r0, EPB)])
        return c
    lax.fori_loop(0, nzb, zcopy, 0)
    plsc.subcore_barrier()

    nch = jnp.minimum(CPT, jnp.maximum(0, N_BLOCKS - tid * CPT))

    def chunk(i, c):
        off = pl.multiple_of((tid * CPT + i) * EPB, 8)
        pltpu.sync_copy(src.at[pl.ds(off, EPB)], srcb)
        pltpu.sync_copy(dst.at[pl.ds(off, EPB)], dstb.at[0])
        pltpu.sync_copy(w.at[pl.ds(off, EPB)], wb)
        pltpu.async_copy(cur.at[srcb], rowsb, sem).wait()

        def mul_group(g, cc):
            base = g * HALF_DIM
            wv = wb[pl.ds(base, HALF_DIM)]
            for k in range(HALF_DIM):
                e = base + k
                splat = jnp.take_along_axis(
                    wv, jnp.full((HALF_DIM,), k, jnp.int32), axis=0,
                    mode="promise_in_bounds")
                rowsb[e, :] = rowsb[e, :] * splat
            return cc
        lax.fori_loop(0, EPB // HALF_DIM, mul_group, 0)
        pltpu.sync_copy(rowsb, acc.at[dstb.at[0]], add=True)
        return c
    lax.fori_loop(0, nch, chunk, 0)
    plsc.subcore_barrier()

    ndb = jnp.minimum(DBPT, jnp.maximum(0, N_DB - tid * DBPT))

    def dcopy(i, c):
        r0 = pl.multiple_of((tid * DBPT + i) * DBLK, 8)
        pltpu.sync_copy(acc.at[pl.ds(r0, DBLK)], out.at[pl.ds(r0, DBLK)])
        return c
    lax.fori_loop(0, ndb, dcopy, 0)
    plsc.subcore_barrier()


def _sc_body(embA, embB, src, dst, w,
             x1A, x2A, x3A, x1B, x2B, x3B,
             srcb, dstb, wb, rowsb, acc, sem):
    cid = lax.axis_index("c")
    tid = lax.axis_index("s")

    def run(cur0, outs):
        cur = cur0
        for out in outs:
            _layer(cur, out, src, dst, w, tid, srcb, dstb, wb, rowsb, acc, sem)
            cur = out

    @pl.when(cid == 0)
    def _():
        run(embA, (x1A, x2A, x3A))

    @pl.when(cid == 1)
    def _():
        run(embB, (x1B, x2B, x3B))


_half = jax.ShapeDtypeStruct((N_NODES, HALF_DIM), jnp.float32)

_sc_call = pl.kernel(
    _sc_body,
    out_type=(_half,) * 6,
    mesh=plsc.VectorSubcoreMesh(core_axis_name="c", subcore_axis_name="s"),
    scratch_types=[
        pltpu.VMEM((EPB,), jnp.int32),
        pltpu.VMEM((1, EPB), jnp.int32),
        pltpu.VMEM((EPB,), jnp.float32),
        pltpu.VMEM((EPB, HALF_DIM), jnp.float32),
        pltpu.VMEM_SHARED((ACC_ROWS, HALF_DIM), jnp.float32),
        pltpu.SemaphoreType.DMA,
    ],
    compiler_params=pltpu.CompilerParams(use_tc_tiling_on_sc=False),
)


_BLK = 1000


def _mean_body(emb, a1, a2, a3, b1, b2, b3, out):
    out[:, :HALF_DIM] = (emb[:, :HALF_DIM] + a1[...] + a2[...] + a3[...]) * 0.25
    out[:, HALF_DIM:] = (emb[:, HALF_DIM:] + b1[...] + b2[...] + b3[...]) * 0.25


_mean_call = pl.pallas_call(
    _mean_body,
    out_shape=jax.ShapeDtypeStruct((N_NODES, EMBED_DIM), jnp.float32),
    grid=(N_NODES // _BLK,),
    in_specs=[pl.BlockSpec((_BLK, EMBED_DIM), lambda i: (i, 0))]
    + [pl.BlockSpec((_BLK, HALF_DIM), lambda i: (i, 0))] * 6,
    out_specs=pl.BlockSpec((_BLK, EMBED_DIM), lambda i: (i, 0)),
)


def kernel(embed, edge_index, edge_weight):
    embA = embed[:, :HALF_DIM]
    embB = embed[:, HALF_DIM:]
    src = edge_index[0]
    dst = edge_index[1]
    x1A, x2A, x3A, x1B, x2B, x3B = _sc_call(embA, embB, src, dst, edge_weight)
    return _mean_call(embed, x1A, x2A, x3A, x1B, x2B, x3B)


# speedup vs baseline: 13.5275x; 3.3254x over previous
import functools

import jax
import jax.numpy as jnp
from jax import lax
from jax.experimental import pallas as pl
from jax.experimental.pallas import tpu as pltpu
from jax.experimental.pallas import tpu_sc as plsc

N_NODES = 100000
EMBED_DIM = 32
HALF_DIM = 16
N_EDGES = 1600000

EPB = 128
N_BLOCKS = N_EDGES // EPB
NS = 16
CPT = -(-N_BLOCKS // NS)
GRP = 8
ACC_ROWS = 100096
N_ZB = ACC_ROWS // EPB
ZBPT = -(-N_ZB // NS)
DBLK = 400
N_DB = N_NODES // DBLK
DBPT = -(-N_DB // NS)


def _mul_block(rowsb, w_sb, b):
    def grp16(g, c):
        wv = w_sb[b, pl.ds(g * HALF_DIM, HALF_DIM)]
        for k in range(HALF_DIM):
            e = g * HALF_DIM + k
            splat = jnp.take_along_axis(
                wv, jnp.full((HALF_DIM,), k, jnp.int32), axis=0,
                mode="promise_in_bounds")
            rowsb[b, e, :] = rowsb[b, e, :] * splat
        return c
    lax.fori_loop(0, EPB // HALF_DIM, grp16, 0)


def _layer(cur, out, src2, dst2, w2, tid,
           src_sb, dst_sb, w_sb, rowsb, acc, sem_e, sem_g, sem_s):
    def zb(e, c):
        rowsb[0, e, :] = jnp.zeros((HALF_DIM,), jnp.float32)
        return c
    lax.fori_loop(0, EPB, zb, 0, unroll=8)
    nzb = jnp.minimum(ZBPT, jnp.maximum(0, N_ZB - tid * ZBPT))

    def zcopy(i, c):
        r0 = pl.multiple_of((tid * ZBPT + i) * EPB, 8)
        pltpu.sync_copy(rowsb.at[0], acc.at[pl.ds(r0, EPB)])
        return c
    lax.fori_loop(0, nzb, zcopy, 0)
    plsc.subcore_barrier()

    nch = jnp.minimum(CPT, jnp.maximum(0, N_BLOCKS - tid * CPT))
    ngrp = nch // GRP

    def group(gi, c):
        blk0 = pl.multiple_of(tid * CPT + gi * GRP, 2)
        cps = [pltpu.async_copy(src2.at[pl.ds(blk0, GRP)], src_sb, sem_e),
               pltpu.async_copy(dst2.at[pl.ds(blk0, GRP)], dst_sb, sem_e),
               pltpu.async_copy(w2.at[pl.ds(blk0, GRP)], w_sb, sem_e)]
        for cp in cps:
            cp.wait()
        gs = [pltpu.async_copy(cur.at[src_sb.at[b]], rowsb.at[b], sem_g)
              for b in range(GRP)]
        for g in gs:
            g.wait()
        for b in range(GRP):
            _mul_block(rowsb, w_sb, b)
        ss = [pltpu.async_copy(rowsb.at[b], acc.at[dst_sb.at[b]], sem_s,
                               add=True)
              for b in range(GRP)]
        for s in ss:
            s.wait()
        return c
    lax.fori_loop(0, ngrp, group, 0)

    def rem(i, c):
        blk = tid * CPT + ngrp * GRP + i
        pltpu.sync_copy(src2.at[pl.ds(blk, 1)], src_sb.at[pl.ds(0, 1)])
        pltpu.sync_copy(dst2.at[pl.ds(blk, 1)], dst_sb.at[pl.ds(0, 1)])
        pltpu.sync_copy(w2.at[pl.ds(blk, 1)], w_sb.at[pl.ds(0, 1)])
        pltpu.async_copy(cur.at[src_sb.at[0]], rowsb.at[0], sem_g).wait()
        _mul_block(rowsb, w_sb, 0)
        pltpu.async_copy(rowsb.at[0], acc.at[dst_sb.at[0]], sem_s,
                         add=True).wait()
        return c
    lax.fori_loop(0, nch - ngrp * GRP, rem, 0)
    plsc.subcore_barrier()

    ndb = jnp.minimum(DBPT, jnp.maximum(0, N_DB - tid * DBPT))

    def dcopy(i, c):
        r0 = pl.multiple_of((tid * DBPT + i) * DBLK, 8)
        pltpu.sync_copy(acc.at[pl.ds(r0, DBLK)], out.at[pl.ds(r0, DBLK)])
        return c
    lax.fori_loop(0, ndb, dcopy, 0)
    plsc.subcore_barrier()


def _sc_body(embA, embB, src2, dst2, w2,
             x1A, x2A, x3A, x1B, x2B, x3B,
             src_sb, dst_sb, w_sb, rowsb, acc, sem_e, sem_g, sem_s):
    cid = lax.axis_index("c")
    tid = lax.axis_index("s")

    def run(cur0, outs):
        cur = cur0
        for out in outs:
            _layer(cur, out, src2, dst2, w2, tid,
                   src_sb, dst_sb, w_sb, rowsb, acc, sem_e, sem_g, sem_s)
            cur = out

    @pl.when(cid == 0)
    def _():
        run(embA, (x1A, x2A, x3A))

    @pl.when(cid == 1)
    def _():
        run(embB, (x1B, x2B, x3B))


_half = jax.ShapeDtypeStruct((N_NODES, HALF_DIM), jnp.float32)

_sc_call = pl.kernel(
    _sc_body,
    out_type=(_half,) * 6,
    mesh=plsc.VectorSubcoreMesh(core_axis_name="c", subcore_axis_name="s"),
    scratch_types=[
        pltpu.VMEM((GRP, EPB), jnp.int32),
        pltpu.VMEM((GRP, EPB), jnp.int32),
        pltpu.VMEM((GRP, EPB), jnp.float32),
        pltpu.VMEM((GRP, EPB, HALF_DIM), jnp.float32),
        pltpu.VMEM_SHARED((ACC_ROWS, HALF_DIM), jnp.float32),
        pltpu.SemaphoreType.DMA,
        pltpu.SemaphoreType.DMA,
        pltpu.SemaphoreType.DMA,
    ],
    compiler_params=pltpu.CompilerParams(use_tc_tiling_on_sc=False),
)


_BLK = 1000


def _mean_body(emb, a1, a2, a3, b1, b2, b3, out):
    out[:, :HALF_DIM] = (emb[:, :HALF_DIM] + a1[...] + a2[...] + a3[...]) * 0.25
    out[:, HALF_DIM:] = (emb[:, HALF_DIM:] + b1[...] + b2[...] + b3[...]) * 0.25


_mean_call = pl.pallas_call(
    _mean_body,
    out_shape=jax.ShapeDtypeStruct((N_NODES, EMBED_DIM), jnp.float32),
    grid=(N_NODES // _BLK,),
    in_specs=[pl.BlockSpec((_BLK, EMBED_DIM), lambda i: (i, 0))]
    + [pl.BlockSpec((_BLK, HALF_DIM), lambda i: (i, 0))] * 6,
    out_specs=pl.BlockSpec((_BLK, EMBED_DIM), lambda i: (i, 0)),
)


def kernel(embed, edge_index, edge_weight):
    embA = embed[:, :HALF_DIM]
    embB = embed[:, HALF_DIM:]
    src2 = edge_index[0].reshape(N_BLOCKS, EPB)
    dst2 = edge_index[1].reshape(N_BLOCKS, EPB)
    w2 = edge_weight.reshape(N_BLOCKS, EPB)
    x1A, x2A, x3A, x1B, x2B, x3B = _sc_call(embA, embB, src2, dst2, w2)
    return _mean_call(embed, x1A, x2A, x3A, x1B, x2B, x3B)


# speedup vs baseline: 14.3580x; 1.0614x over previous
import functools

import jax
import jax.numpy as jnp
from jax import lax
from jax.experimental import pallas as pl
from jax.experimental.pallas import tpu as pltpu
from jax.experimental.pallas import tpu_sc as plsc

N_NODES = 100000
EMBED_DIM = 32
HALF_DIM = 16
N_EDGES = 1600000

EPB = 128
N_BLOCKS = N_EDGES // EPB
NS = 16
CPT = -(-N_BLOCKS // NS)
GRP = 8
ACC_ROWS = 100096
N_ZB = ACC_ROWS // EPB
ZBPT = -(-N_ZB // NS)
DBLK = 200
N_DB = N_NODES // DBLK
DBPT = -(-N_DB // NS)


def _mul_block(rowsb, w1, b, woff):
    def grp16(g, c):
        wv = w1[pl.ds(woff + g * HALF_DIM, HALF_DIM)]
        for k in range(HALF_DIM):
            e = b * EPB + g * HALF_DIM + k
            splat = jnp.take_along_axis(
                wv, jnp.full((HALF_DIM,), k, jnp.int32), axis=0,
                mode="promise_in_bounds")
            rowsb[e, :] = rowsb[e, :] * splat
        return c
    lax.fori_loop(0, EPB // HALF_DIM, grp16, 0)


def _layer(cur, out, src, dst, w, tid, last,
           srcb, dstb, wb, rowsb, x1, x2,
           acc, sem_e, sem_g, sem_s):
    def zb(e, c):
        rowsb[e, :] = jnp.zeros((HALF_DIM,), jnp.float32)
        return c
    lax.fori_loop(0, EPB, zb, 0, unroll=8)
    nzb = jnp.minimum(ZBPT, jnp.maximum(0, N_ZB - tid * ZBPT))

    def zcopy(i, c):
        r0 = pl.multiple_of((tid * ZBPT + i) * EPB, 8)
        pltpu.async_copy(rowsb.at[pl.ds(0, EPB)], acc.at[pl.ds(r0, EPB)], sem_s).wait()
        return c
    lax.fori_loop(0, nzb, zcopy, 0)
    plsc.subcore_barrier()

    nch = jnp.minimum(CPT, jnp.maximum(0, N_BLOCKS - tid * CPT))
    ngrp = nch // GRP

    def group(gi, c):
        off = pl.multiple_of((tid * CPT + gi * GRP) * EPB, 8)
        cps = [pltpu.async_copy(src.at[pl.ds(off, GRP * EPB)], srcb, sem_e),
               pltpu.async_copy(w.at[pl.ds(off, GRP * EPB)], wb, sem_e)]
        cps += [pltpu.async_copy(dst.at[pl.ds(off + b * EPB, EPB)],
                                 dstb.at[b], sem_e)
                for b in range(GRP)]
        for cp in cps:
            cp.wait()
        gs = [pltpu.async_copy(cur.at[srcb.at[pl.ds(b * EPB, EPB)]],
                               rowsb.at[pl.ds(b * EPB, EPB)], sem_g)
              for b in range(GRP)]
        for g in gs:
            g.wait()
        for b in range(GRP):
            _mul_block(rowsb, wb, b, b * EPB)
        ss = [pltpu.async_copy(rowsb.at[pl.ds(b * EPB, EPB)],
                               acc.at[dstb.at[b]], sem_s, add=True)
              for b in range(GRP)]
        for s in ss:
            s.wait()
        return c
    lax.fori_loop(0, ngrp, group, 0)

    def rem(i, c):
        off = pl.multiple_of((tid * CPT + ngrp * GRP + i) * EPB, 8)
        pltpu.sync_copy(src.at[pl.ds(off, EPB)], srcb.at[pl.ds(0, EPB)])
        pltpu.sync_copy(dst.at[pl.ds(off, EPB)], dstb.at[0])
        pltpu.sync_copy(w.at[pl.ds(off, EPB)], wb.at[pl.ds(0, EPB)])
        pltpu.async_copy(cur.at[srcb.at[pl.ds(0, EPB)]],
                         rowsb.at[pl.ds(0, EPB)], sem_g).wait()
        _mul_block(rowsb, wb, 0, 0)
        pltpu.async_copy(rowsb.at[pl.ds(0, EPB)], acc.at[dstb.at[0]], sem_s,
                         add=True).wait()
        return c
    lax.fori_loop(0, nch - ngrp * GRP, rem, 0)
    plsc.subcore_barrier()

    ndb = jnp.minimum(DBPT, jnp.maximum(0, N_DB - tid * DBPT))

    if not last:
        def dcopy(i, c):
            r0 = pl.multiple_of((tid * DBPT + i) * DBLK, 8)
            pltpu.async_copy(acc.at[pl.ds(r0, DBLK)], out.at[pl.ds(r0, DBLK)],
                             sem_s).wait()
            return c
        lax.fori_loop(0, ndb, dcopy, 0)
    else:
        def dsum(i, c):
            r0 = pl.multiple_of((tid * DBPT + i) * DBLK, 8)
            c1 = pltpu.async_copy(x1.at[pl.ds(r0, DBLK)],
                                  rowsb.at[pl.ds(0, DBLK)], sem_e)
            c2 = pltpu.async_copy(x2.at[pl.ds(r0, DBLK)],
                                  rowsb.at[pl.ds(DBLK, DBLK)], sem_e)
            c3 = pltpu.async_copy(acc.at[pl.ds(r0, DBLK)],
                                  rowsb.at[pl.ds(2 * DBLK, DBLK)], sem_g)
            c1.wait(); c2.wait(); c3.wait()

            def addrow(r, cc):
                rowsb[r, :] = (rowsb[r, :] + rowsb[DBLK + r, :]
                               + rowsb[2 * DBLK + r, :])
                return cc
            lax.fori_loop(0, DBLK, addrow, 0, unroll=8)
            pltpu.async_copy(rowsb.at[pl.ds(0, DBLK)],
                             out.at[pl.ds(r0, DBLK)], sem_s).wait()
            return c
        lax.fori_loop(0, ndb, dsum, 0)
    plsc.subcore_barrier()


def _sc_body(embA, embB, src, dst, w,
             x1A, x2A, sumA, x1B, x2B, sumB,
             srcb, dstb, wb, rowsb,
             acc, sem_e, sem_g, sem_s):
    cid = lax.axis_index("c")
    tid = lax.axis_index("s")

    def run(emb, x1, x2, out):
        _layer(emb, x1, src, dst, w, tid, False,
               srcb, dstb, wb, rowsb, x1, x2,
               acc, sem_e, sem_g, sem_s)
        _layer(x1, x2, src, dst, w, tid, False,
               srcb, dstb, wb, rowsb, x1, x2,
               acc, sem_e, sem_g, sem_s)
        _layer(x2, out, src, dst, w, tid, True,
               srcb, dstb, wb, rowsb, x1, x2,
               acc, sem_e, sem_g, sem_s)

    @pl.when(cid == 0)
    def _():
        run(embA, x1A, x2A, sumA)

    @pl.when(cid == 1)
    def _():
        run(embB, x1B, x2B, sumB)


_half = jax.ShapeDtypeStruct((N_NODES, HALF_DIM), jnp.float32)

_sc_call = pl.kernel(
    _sc_body,
    out_type=(_half,) * 6,
    mesh=plsc.VectorSubcoreMesh(core_axis_name="c", subcore_axis_name="s"),
    scratch_types=[
        pltpu.VMEM((GRP * EPB,), jnp.int32),
        pltpu.VMEM((GRP, EPB), jnp.int32),
        pltpu.VMEM((GRP * EPB,), jnp.float32),
        pltpu.VMEM((GRP * EPB, HALF_DIM), jnp.float32),
        pltpu.VMEM_SHARED((ACC_ROWS, HALF_DIM), jnp.float32),
        pltpu.SemaphoreType.DMA,
        pltpu.SemaphoreType.DMA,
        pltpu.SemaphoreType.DMA,
    ],
    compiler_params=pltpu.CompilerParams(use_tc_tiling_on_sc=False),
)


_BLK = 1000


def _mean_body(emb, sa, sb, out):
    out[:, :HALF_DIM] = (emb[:, :HALF_DIM] + sa[...]) * 0.25
    out[:, HALF_DIM:] = (emb[:, HALF_DIM:] + sb[...]) * 0.25


_mean_call = pl.pallas_call(
    _mean_body,
    out_shape=jax.ShapeDtypeStruct((N_NODES, EMBED_DIM), jnp.float32),
    grid=(N_NODES // _BLK,),
    in_specs=[pl.BlockSpec((_BLK, EMBED_DIM), lambda i: (i, 0))]
    + [pl.BlockSpec((_BLK, HALF_DIM), lambda i: (i, 0))] * 2,
    out_specs=pl.BlockSpec((_BLK, EMBED_DIM), lambda i: (i, 0)),
)


def kernel(embed, edge_index, edge_weight):
    embA = embed[:, :HALF_DIM]
    embB = embed[:, HALF_DIM:]
    src = edge_index[0]
    dst = edge_index[1]
    _, _, sumA, _, _, sumB = _sc_call(embA, embB, src, dst, edge_weight)
    return _mean_call(embed, sumA, sumB)


# speedup vs baseline: 16.2281x; 1.1302x over previous
import functools

import jax
import jax.numpy as jnp
from jax import lax
from jax.experimental import pallas as pl
from jax.experimental.pallas import tpu as pltpu
from jax.experimental.pallas import tpu_sc as plsc

N_NODES = 100000
EMBED_DIM = 32
HALF_DIM = 16
N_EDGES = 1600000

EPB = 128
N_BLOCKS = N_EDGES // EPB
NS = 16
CPT = -(-N_BLOCKS // NS)
GRP = 4
PPG = GRP * EPB
DBLK = 200
N_DB = N_NODES // DBLK
DBPT = -(-N_DB // NS)


def _mul_block(rowsb, wb, roff):
    def grp16(g, c):
        base = roff + g * HALF_DIM
        wv = wb[pl.ds(base, HALF_DIM)]
        for k in range(HALF_DIM):
            splat = jnp.take_along_axis(
                wv, jnp.full((HALF_DIM,), k, jnp.int32), axis=0,
                mode="promise_in_bounds")
            rowsb[base + k, :] = rowsb[base + k, :] * splat
        return c
    lax.fori_loop(0, EPB // HALF_DIM, grp16, 0)


def _layer(cid, tid, cur, out, src, dst, w, last, x1, x2,
           srcb, dstb, wb, rowsb, acc, sem_e, sem_g, sem_s):
    def zb(e, c):
        rowsb[e, :] = jnp.zeros((HALF_DIM,), jnp.float32)
        return c
    lax.fori_loop(0, DBLK, zb, 0, unroll=8)
    nzb = jnp.minimum(DBPT, jnp.maximum(0, N_DB - tid * DBPT))

    def zissue(i, c):
        r0 = pl.multiple_of((tid * DBPT + i) * DBLK, 8)
        pltpu.async_copy(rowsb.at[pl.ds(0, DBLK)], acc.at[pl.ds(r0, DBLK)],
                         sem_s)
        return c
    lax.fori_loop(0, nzb, zissue, 0)

    def zdrain(i, c):
        pltpu.make_async_copy(rowsb.at[pl.ds(0, DBLK)],
                              acc.at[pl.ds(0, DBLK)], sem_s).wait()
        return c
    lax.fori_loop(0, nzb, zdrain, 0)
    plsc.subcore_barrier()

    nch = jnp.minimum(CPT, jnp.maximum(0, N_BLOCKS - tid * CPT))
    ngrp = nch // GRP
    npair = ngrp // 2

    def off_of(g):
        blk = jnp.minimum(tid * CPT + g * GRP, N_BLOCKS - GRP)
        return pl.multiple_of(blk * EPB, 8)

    def issue_edges(g, p):
        off = off_of(g)
        pltpu.async_copy(src.at[pl.ds(off, PPG)],
                         srcb.at[pl.ds(p * PPG, PPG)], sem_e)
        pltpu.async_copy(w.at[pl.ds(off, PPG)],
                         wb.at[pl.ds(p * PPG, PPG)], sem_e)
        for b in range(GRP):
            pltpu.async_copy(dst.at[pl.ds(off + b * EPB, EPB)],
                             dstb.at[p * GRP + b], sem_e)

    def drain_edges():
        pltpu.make_async_copy(src.at[pl.ds(0, PPG)],
                              srcb.at[pl.ds(0, PPG)], sem_e).wait()
        pltpu.make_async_copy(w.at[pl.ds(0, PPG)],
                              wb.at[pl.ds(0, PPG)], sem_e).wait()
        for _ in range(GRP):
            pltpu.make_async_copy(dst.at[pl.ds(0, EPB)], dstb.at[0],
                                  sem_e).wait()

    def issue_gathers(p):
        for b in range(GRP):
            o = p * PPG + b * EPB
            pltpu.async_copy(cur.at[cid].at[srcb.at[pl.ds(o, EPB)]],
                             rowsb.at[pl.ds(o, EPB)], sem_g)

    def drain_gathers():
        for _ in range(GRP):
            pltpu.make_async_copy(cur.at[cid, pl.ds(0, EPB)],
                                  rowsb.at[pl.ds(0, EPB)], sem_g).wait()

    def issue_scatters(p):
        for b in range(GRP):
            o = p * PPG + b * EPB
            pltpu.async_copy(rowsb.at[pl.ds(o, EPB)],
                             acc.at[dstb.at[p * GRP + b]], sem_s, add=True)

    def drain_scatters():
        for _ in range(GRP):
            pltpu.make_async_copy(rowsb.at[pl.ds(0, EPB)],
                                  acc.at[pl.ds(0, EPB)], sem_s).wait()

    issue_edges(0, 0)
    drain_edges()
    issue_gathers(0)

    def pair(i, c):
        for p in (0, 1):
            drain_gathers()
            if p == 0:
                @pl.when(i > 0)
                def _():
                    drain_scatters()
            else:
                drain_scatters()
            issue_edges(2 * i + p + 1, p ^ 1)
            for b in range(GRP):
                _mul_block(rowsb, wb, p * PPG + b * EPB)
            issue_scatters(p)
            drain_edges()
            issue_gathers(p ^ 1)
        return c
    lax.fori_loop(0, npair, pair, 0)
    drain_gathers()
    drain_scatters()

    def rem(i, c):
        off = pl.multiple_of((tid * CPT + npair * 2 * GRP + i) * EPB, 8)
        pltpu.sync_copy(src.at[pl.ds(off, EPB)], srcb.at[pl.ds(0, EPB)])
        pltpu.sync_copy(dst.at[pl.ds(off, EPB)], dstb.at[0])
        pltpu.sync_copy(w.at[pl.ds(off, EPB)], wb.at[pl.ds(0, EPB)])
        pltpu.async_copy(cur.at[cid].at[srcb.at[pl.ds(0, EPB)]],
                         rowsb.at[pl.ds(0, EPB)], sem_g).wait()
        _mul_block(rowsb, wb, 0)
        pltpu.async_copy(rowsb.at[pl.ds(0, EPB)], acc.at[dstb.at[0]], sem_s,
                         add=True).wait()
        return c
    lax.fori_loop(0, nch - npair * 2 * GRP, rem, 0)
    plsc.subcore_barrier()

    ndb = jnp.minimum(DBPT, jnp.maximum(0, N_DB - tid * DBPT))

    if not last:
        def dissue(i, c):
            r0 = pl.multiple_of((tid * DBPT + i) * DBLK, 8)
            pltpu.async_copy(acc.at[pl.ds(r0, DBLK)],
                             out.at[cid, pl.ds(r0, DBLK)], sem_s)
            return c
        lax.fori_loop(0, ndb, dissue, 0)

        def ddrain(i, c):
            pltpu.make_async_copy(acc.at[pl.ds(0, DBLK)],
                                  out.at[cid, pl.ds(0, DBLK)], sem_s).wait()
            return c
        lax.fori_loop(0, ndb, ddrain, 0)
    else:
        def dsum(i, c):
            r0 = pl.multiple_of((tid * DBPT + i) * DBLK, 8)
            c1 = pltpu.async_copy(x1.at[cid, pl.ds(r0, DBLK)],
                                  rowsb.at[pl.ds(0, DBLK)], sem_e)
            c2 = pltpu.async_copy(x2.at[cid, pl.ds(r0, DBLK)],
                                  rowsb.at[pl.ds(DBLK, DBLK)], sem_e)
            c3 = pltpu.async_copy(acc.at[pl.ds(r0, DBLK)],
                                  rowsb.at[pl.ds(2 * DBLK, DBLK)], sem_g)
            c1.wait(); c2.wait(); c3.wait()

            def addrow(r, cc):
                rowsb[r, :] = (rowsb[r, :] + rowsb[DBLK + r, :]
                               + rowsb[2 * DBLK + r, :])
                return cc
            lax.fori_loop(0, DBLK, addrow, 0, unroll=8)
            pltpu.async_copy(rowsb.at[pl.ds(0, DBLK)],
                             out.at[cid, pl.ds(r0, DBLK)], sem_s).wait()
            return c
        lax.fori_loop(0, ndb, dsum, 0)
    plsc.subcore_barrier()


def _sc_body(emb2, src, dst, w, x1s, x2s, sums,
             srcb, dstb, wb, rowsb, acc, sem_e, sem_g, sem_s):
    cid = lax.axis_index("c")
    tid = lax.axis_index("s")
    _layer(cid, tid, emb2, x1s, src, dst, w, False, x1s, x2s,
           srcb, dstb, wb, rowsb, acc, sem_e, sem_g, sem_s)
    _layer(cid, tid, x1s, x2s, src, dst, w, False, x1s, x2s,
           srcb, dstb, wb, rowsb, acc, sem_e, sem_g, sem_s)
    _layer(cid, tid, x2s, sums, src, dst, w, True, x1s, x2s,
           srcb, dstb, wb, rowsb, acc, sem_e, sem_g, sem_s)


_stk = jax.ShapeDtypeStruct((2, N_NODES, HALF_DIM), jnp.float32)

_sc_call = pl.kernel(
    _sc_body,
    out_type=(_stk,) * 3,
    mesh=plsc.VectorSubcoreMesh(core_axis_name="c", subcore_axis_name="s"),
    scratch_types=[
        pltpu.VMEM((2 * PPG,), jnp.int32),
        pltpu.VMEM((2 * GRP, EPB), jnp.int32),
        pltpu.VMEM((2 * PPG,), jnp.float32),
        pltpu.VMEM((2 * PPG, HALF_DIM), jnp.float32),
        pltpu.VMEM_SHARED((N_NODES, HALF_DIM), jnp.float32),
        pltpu.SemaphoreType.DMA,
        pltpu.SemaphoreType.DMA,
        pltpu.SemaphoreType.DMA,
    ],
    compiler_params=pltpu.CompilerParams(use_tc_tiling_on_sc=False),
)


_BLK = 1000


def _mean_body(emb, sums, out):
    out[:, :HALF_DIM] = (emb[:, :HALF_DIM] + sums[0]) * 0.25
    out[:, HALF_DIM:] = (emb[:, HALF_DIM:] + sums[1]) * 0.25


_mean_call = pl.pallas_call(
    _mean_body,
    out_shape=jax.ShapeDtypeStruct((N_NODES, EMBED_DIM), jnp.float32),
    grid=(N_NODES // _BLK,),
    in_specs=[pl.BlockSpec((_BLK, EMBED_DIM), lambda i: (i, 0)),
              pl.BlockSpec((2, _BLK, HALF_DIM), lambda i: (0, i, 0))],
    out_specs=pl.BlockSpec((_BLK, EMBED_DIM), lambda i: (i, 0)),
)


def kernel(embed, edge_index, edge_weight):
    emb2 = jnp.stack([embed[:, :HALF_DIM], embed[:, HALF_DIM:]], axis=0)
    src = edge_index[0]
    dst = edge_index[1]
    _, _, sums = _sc_call(emb2, src, dst, edge_weight)
    return _mean_call(embed, sums)
